# initial kernel scaffold (unmeasured)
import jax
import jax.numpy as jnp
from jax import lax
from jax.experimental import pallas as pl
from jax.experimental.pallas import tpu as pltpu

N_DEV = 32


def kernel(x, w_mat):
    m_per, k = x.shape
    n = w_mat.shape[1]
    n_per = n // N_DEV
    m_tot = m_per * N_DEV

    def body(x_ref, w_ref, out_ref, y_ref, amax_ref, amax_rx_ref,
             blk_send_sems, blk_recv_sems, amax_send_sems, amax_recv_sems):
        my = lax.axis_index("i")

        barrier = pltpu.get_barrier_semaphore()
        for d in range(1, N_DEV):
            pl.semaphore_signal(
                barrier, inc=1,
                device_id=((my + d) % N_DEV,),
                device_id_type=pl.DeviceIdType.MESH,
            )
        pl.semaphore_wait(barrier, N_DEV - 1)

        y = jnp.dot(x_ref[:, :], w_ref[:, :],
                    preferred_element_type=jnp.float32)
        for j in range(N_DEV):
            y_ref[j, :, :] = y[:, j * n_per:(j + 1) * n_per]
        local_amax = jnp.max(jnp.abs(y))
        amax_ref[:, :] = jnp.full((8, 128), local_amax, jnp.float32)

        sends = []
        for d in range(1, N_DEV):
            tgt = (my + d) % N_DEV
            blk = pltpu.make_async_remote_copy(
                src_ref=y_ref.at[tgt],
                dst_ref=out_ref.at[pl.ds(my * m_per, m_per), :],
                send_sem=blk_send_sems.at[tgt],
                recv_sem=blk_recv_sems.at[my],
                device_id=(tgt,),
                device_id_type=pl.DeviceIdType.MESH,
            )
            blk.start()
            am = pltpu.make_async_remote_copy(
                src_ref=amax_ref,
                dst_ref=amax_rx_ref.at[my],
                send_sem=amax_send_sems.at[tgt],
                recv_sem=amax_recv_sems.at[my],
                device_id=(tgt,),
                device_id_type=pl.DeviceIdType.MESH,
            )
            am.start()
            sends.append((blk, am))

        out_ref[pl.ds(my * m_per, m_per), :] = y_ref[my, :, :]
        amax_rx_ref[pl.ds(my, 1), :, :] = amax_ref[:, :].reshape(1, 8, 128)

        for d in range(1, N_DEV):
            src = (my + d) % N_DEV
            am_rx = pltpu.make_async_remote_copy(
                src_ref=amax_ref,
                dst_ref=amax_rx_ref.at[src],
                send_sem=amax_send_sems.at[src],
                recv_sem=amax_recv_sems.at[src],
                device_id=(src,),
                device_id_type=pl.DeviceIdType.MESH,
            )
            am_rx.wait_recv()
        for d in range(1, N_DEV):
            src = (my + d) % N_DEV
            blk_rx = pltpu.make_async_remote_copy(
                src_ref=y_ref.at[0],
                dst_ref=out_ref.at[pl.ds(src * m_per, m_per), :],
                send_sem=blk_send_sems.at[src],
                recv_sem=blk_recv_sems.at[src],
                device_id=(src,),
                device_id_type=pl.DeviceIdType.MESH,
            )
            blk_rx.wait_recv()

        gmax = jnp.max(amax_rx_ref[:, :, :])
        scale = gmax / 448.0
        q = jnp.clip(out_ref[:, :] / scale, -448.0, 448.0)
        snapped = q.astype(jnp.float8_e4m3fn).astype(jnp.float32)
        out_ref[:, :] = snapped * scale

        for blk, am in sends:
            blk.wait_send()
            am.wait_send()

    return pl.pallas_call(
        body,
        out_shape=jax.ShapeDtypeStruct((m_tot, n_per), jnp.float32),
        in_specs=[
            pl.BlockSpec(memory_space=pltpu.VMEM),
            pl.BlockSpec(memory_space=pltpu.VMEM),
        ],
        out_specs=pl.BlockSpec(memory_space=pltpu.VMEM),
        scratch_shapes=[
            pltpu.VMEM((N_DEV, m_per, n_per), jnp.float32),
            pltpu.VMEM((8, 128), jnp.float32),
            pltpu.VMEM((N_DEV, 8, 128), jnp.float32),
            pltpu.SemaphoreType.DMA((N_DEV,)),
            pltpu.SemaphoreType.DMA((N_DEV,)),
            pltpu.SemaphoreType.DMA((N_DEV,)),
            pltpu.SemaphoreType.DMA((N_DEV,)),
        ],
        compiler_params=pltpu.CompilerParams(collective_id=0),
    )(x, w_mat)


# baseline (device time: 54238 ns/iter reference)
import jax
import jax.numpy as jnp
from jax import lax
from jax.experimental import pallas as pl
from jax.experimental.pallas import tpu as pltpu

N_DEV = 32


def kernel(x, w_mat):
    m_per, k = x.shape
    n = w_mat.shape[1]
    n_per = n // N_DEV
    m_tot = m_per * N_DEV

    def body(x_ref, w_ref, out_ref, y_ref, amax_ref, amax_rx_ref,
             blk_send_sems, blk_recv_sems, amax_send_sems, amax_recv_sems):
        my = lax.axis_index("i")

        barrier = pltpu.get_barrier_semaphore()
        for d in range(1, N_DEV):
            pl.semaphore_signal(
                barrier, inc=1,
                device_id=((my + d) % N_DEV,),
                device_id_type=pl.DeviceIdType.MESH,
            )
        pl.semaphore_wait(barrier, N_DEV - 1)

        y = jnp.dot(x_ref[:, :], w_ref[:, :],
                    preferred_element_type=jnp.float32)
        for j in range(N_DEV):
            y_ref[j, :, :] = y[:, j * n_per:(j + 1) * n_per]
        local_amax = jnp.max(jnp.abs(y))
        amax_ref[:, :] = jnp.full((8, 128), local_amax, jnp.float32)

        sends = []
        for d in range(1, N_DEV):
            tgt = (my + d) % N_DEV
            blk = pltpu.make_async_remote_copy(
                src_ref=y_ref.at[tgt],
                dst_ref=out_ref.at[pl.ds(my * m_per, m_per), :],
                send_sem=blk_send_sems.at[tgt],
                recv_sem=blk_recv_sems.at[my],
                device_id=(tgt,),
                device_id_type=pl.DeviceIdType.MESH,
            )
            blk.start()
            am = pltpu.make_async_remote_copy(
                src_ref=amax_ref,
                dst_ref=amax_rx_ref.at[my],
                send_sem=amax_send_sems.at[tgt],
                recv_sem=amax_recv_sems.at[my],
                device_id=(tgt,),
                device_id_type=pl.DeviceIdType.MESH,
            )
            am.start()
            sends.append((blk, am))

        out_ref[pl.ds(my * m_per, m_per), :] = y_ref[my, :, :]
        amax_rx_ref[pl.ds(my, 1), :, :] = amax_ref[:, :].reshape(1, 8, 128)

        for d in range(1, N_DEV):
            src = (my + d) % N_DEV
            am_rx = pltpu.make_async_remote_copy(
                src_ref=amax_ref,
                dst_ref=amax_rx_ref.at[src],
                send_sem=amax_send_sems.at[src],
                recv_sem=amax_recv_sems.at[src],
                device_id=(src,),
                device_id_type=pl.DeviceIdType.MESH,
            )
            am_rx.wait_recv()
        for d in range(1, N_DEV):
            src = (my + d) % N_DEV
            blk_rx = pltpu.make_async_remote_copy(
                src_ref=y_ref.at[0],
                dst_ref=out_ref.at[pl.ds(src * m_per, m_per), :],
                send_sem=blk_send_sems.at[src],
                recv_sem=blk_recv_sems.at[src],
                device_id=(src,),
                device_id_type=pl.DeviceIdType.MESH,
            )
            blk_rx.wait_recv()

        gmax = jnp.max(amax_rx_ref[:, :, :])
        scale = gmax / 448.0
        q = jnp.clip(out_ref[:, :] / scale, -448.0, 448.0)
        snapped = q.astype(jnp.float8_e4m3fn).astype(jnp.float32)
        out_ref[:, :] = snapped * scale

        for blk, am in sends:
            blk.wait_send()
            am.wait_send()

    return pl.pallas_call(
        body,
        out_shape=jax.ShapeDtypeStruct((m_tot, n_per), jnp.float32),
        in_specs=[
            pl.BlockSpec(memory_space=pltpu.VMEM),
            pl.BlockSpec(memory_space=pltpu.VMEM),
        ],
        out_specs=pl.BlockSpec(memory_space=pltpu.VMEM),
        scratch_shapes=[
            pltpu.VMEM((N_DEV, m_per, n_per), jnp.float32),
            pltpu.VMEM((8, 128), jnp.float32),
            pltpu.VMEM((N_DEV, 8, 128), jnp.float32),
            pltpu.SemaphoreType.DMA((N_DEV,)),
            pltpu.SemaphoreType.DMA((N_DEV,)),
            pltpu.SemaphoreType.DMA((N_DEV,)),
            pltpu.SemaphoreType.DMA((N_DEV,)),
        ],
        compiler_params=pltpu.CompilerParams(
            collective_id=0, vmem_limit_bytes=100 * 1024 * 1024
        ),
    )(x, w_mat)
